# Initial kernel scaffold; baseline (speedup 1.0000x reference)
import jax
import jax.numpy as jnp
from jax.experimental import pallas as pl


def kernel(h_V_s, h_V_v, edge_index, h_E_s, h_E_v, params):
    raise NotImplementedError("write your pallas kernel here")


# speedup vs baseline: 10.9423x; 10.9423x over previous
import functools

import numpy as np
import jax
import jax.numpy as jnp
from jax import lax
from jax.experimental import pallas as pl
from jax.experimental.pallas import tpu as pltpu
from jax.experimental.pallas import tpu_sc as plsc

_N = 3200
_E = 51200
_B = 64
_NODE = 50
_W = 160
_EF = 35
_F32 = jnp.float32

_GW = 128
_SW = 80
_CW = 80


def _bd3(w):
    return jnp.kron(jnp.eye(3, dtype=w.dtype), w)


def _gvp_mats(p):
    wh = p["wh"]
    ws = p["ws_w"]
    h = wh.shape[1]
    si = ws.shape[0] - h
    out = {
        "WH": _bd3(wh),
        "WSs": ws[:si],
        "WSv": ws[si:],
        "b": p["ws_b"][None, :],
    }
    if "wv" in p:
        out["WV"] = _bd3(p["wv"])
    return out


def _msg_prep(p):
    g0 = _gvp_mats(p["msg0"])
    rs = np.array([a * 33 + c for a in range(3) for c in range(16)])
    re = np.array([a * 33 + 16 for a in range(3)])
    rd = np.array([a * 33 + 17 + c for a in range(3) for c in range(16)])
    A_src = (jnp.zeros((_W, 199), _F32)
             .at[0:100, 0:100].set(g0["WSs"][0:100])
             .at[100:148, 100:199].set(g0["WH"][rs]))
    A_dst = (jnp.zeros((_W, 199), _F32)
             .at[0:100, 0:100].set(g0["WSs"][132:232])
             .at[100:148, 100:199].set(g0["WH"][rd]))
    A_e = (jnp.zeros((_EF, 199), _F32)
           .at[0:32, 0:100].set(g0["WSs"][100:132])
           .at[32:35, 100:199].set(g0["WH"][re]))
    b0 = jnp.zeros((1, 199), _F32).at[:, 0:100].set(g0["b"])

    def mid(gk):
        M = (jnp.zeros((148, 148), _F32)
             .at[0:100, 0:100].set(gk["WSs"])
             .at[100:148, 100:148].set(gk["WH"]))
        b = jnp.zeros((1, 148), _F32).at[:, 0:100].set(gk["b"])
        return M, b

    g1 = _gvp_mats(p["msg1"])
    g2 = _gvp_mats(p["msg2"])
    M1, b1 = mid(g1)
    M2, b2 = mid(g2)
    return (A_src, A_dst, A_e, b0, g0["WSv"], g0["WV"],
            M1, b1, g1["WSv"], g1["WV"],
            M2, b2, g2["WSv"], g2["WV"])


def _ff_prep(p):
    f0 = _gvp_mats(p["ff0"])
    f1 = _gvp_mats(p["ff1"])
    F0 = (jnp.zeros((148, 496), _F32)
          .at[0:100, 0:400].set(f0["WSs"])
          .at[100:148, 400:496].set(f0["WH"]))
    f0b = jnp.zeros((1, 496), _F32).at[:, 0:400].set(f0["b"])
    F1 = (jnp.zeros((496, 196), _F32)
          .at[0:400, 0:100].set(f1["WSs"])
          .at[400:496, 100:196].set(f1["WH"]))
    f1b = jnp.zeros((1, 196), _F32).at[:, 0:100].set(f1["b"])
    n0, n1 = p["norm0"], p["norm1"]
    return (n0["g"][None, :], n0["b"][None, :], F0, f0b, f0["WSv"], f0["WV"],
            F1, f1b, f1["WSv"], f1["WV"], n1["g"][None, :], n1["b"][None, :])


def _prep(params):
    wv = _gvp_mats(params["W_v"])
    wvln = params["W_v_ln"]
    we = params["W_e"]
    weln = params["W_e_ln"]
    wo = _gvp_mats(params["W_out"])

    perm = np.array([n * 48 + c * 3 + a
                     for n in range(_NODE) for a in range(3) for c in range(16)])
    (q0w, q0b), (q1w, q1b), (q2w, q2b) = params["sq"]
    (u0w, u0b), (u1w, u1b), (u2w, u2b) = params["us"]
    mlp = (q0w[:5000], q0w[5000:][perm], q0b[None, :],
           q1w, q1b[None, :], q2w, q2b[None, :],
           u0w, u0b[None, :], u1w, u1b[None, :],
           u2w[:, :5000], u2w[:, 5000:][:, perm],
           u2b[None, :5000], u2b[None, 5000:][:, perm])

    return {
        "W_v": (wv["WH"], wv["WSs"], wv["WSv"], wv["b"], wv["WV"],
                wvln["g"][None, :], wvln["b"][None, :]),
        "W_e": (we["wh"], we["ws_w"][:32], we["ws_w"][32:33], we["ws_b"][None, :],
                we["wv"], weln["g"][None, :], weln["b"][None, :]),
        "enc": [{"msg": _msg_prep(p), "ff": _ff_prep(p)} for p in params["enc"]],
        "dec": [{"msg": _msg_prep(p), "ff": _ff_prep(p)} for p in params["dec"]],
        "mlp": mlp,
        "W_out": (wo["WH"], wo["WSs"], wo["WSv"], wo["b"]),
    }


def _dot(a, b):
    return jnp.dot(a, b, preferred_element_type=_F32)


def _norm3(x):
    h = x.shape[-1] // 3
    sq = x[:, :h] * x[:, :h] + x[:, h:2 * h] * x[:, h:2 * h] + x[:, 2 * h:] * x[:, 2 * h:]
    return jnp.sqrt(jnp.maximum(sq, 1e-8))


def _gate(v):
    g = jax.nn.sigmoid(_norm3(v))
    return v * jnp.concatenate([g, g, g], axis=1)


def _ln_math(s, v, g, b):
    mu = jnp.mean(s, axis=-1, keepdims=True)
    var = jnp.mean((s - mu) ** 2, axis=-1, keepdims=True)
    s = (s - mu) / jnp.sqrt(var + 1e-5) * g + b
    c = v.shape[-1] // 3
    sq = v[:, :c] * v[:, :c] + v[:, c:2 * c] * v[:, c:2 * c] + v[:, 2 * c:] * v[:, 2 * c:]
    vn = jnp.sqrt(jnp.mean(jnp.maximum(sq, 1e-8), axis=-1, keepdims=True))
    return s, v / vn


def _pad12(x):
    return jnp.concatenate([x, jnp.zeros((x.shape[0], _W - 148), _F32)], axis=1)


def _embed_math(s6, v9, c0, c1, w):
    WH, WSs, WSv, b, WV, lg, lb = w
    vh = _dot(v9, WH)
    s = _dot(s6, WSs) + _dot(_norm3(vh), WSv) + b
    v = _dot(vh, WV)
    s, v = _ln_math(s, v, lg, lb)
    inv = 1.0 / jnp.maximum(c0[:, 0:1] + c1[:, 0:1], 1.0)
    return _pad12(jnp.concatenate([s, v], axis=1)), inv


def _edge_math(es, ev, w):
    whs, WSs, wsv, b, wvs, lg, lb = w
    vh = ev * whs[0, 0]
    vn = jnp.sqrt(jnp.maximum(jnp.sum(vh * vh, axis=1, keepdims=True), 1e-8))
    s = _dot(es, WSs) + vn * wsv + b
    vo = vh * wvs[0, 0]
    mu = jnp.mean(s, axis=-1, keepdims=True)
    var = jnp.mean((s - mu) ** 2, axis=-1, keepdims=True)
    s = (s - mu) / jnp.sqrt(var + 1e-5) * lg + lb
    sq = jnp.sum(vo * vo, axis=1, keepdims=True)
    vo = vo / jnp.sqrt(jnp.maximum(sq, 1e-8))
    return jnp.concatenate([s, vo], axis=1)


def _msg_math(gs, gd, ef, w):
    (A_src, A_dst, A_e, b0, S0v, WV0, M1, b1, S1v, WV1, M2, b2, S2v, WV2) = w
    X = _dot(gs, A_src) + _dot(gd, A_dst) + _dot(ef, A_e) + b0
    vh = X[:, 100:]
    s = jax.nn.relu(X[:, :100] + _dot(_norm3(vh), S0v))
    v = _gate(_dot(vh, WV0))
    Y = _dot(jnp.concatenate([s, v], axis=1), M1) + b1
    vh = Y[:, 100:]
    s = jax.nn.relu(Y[:, :100] + _dot(_norm3(vh), S1v))
    v = _gate(_dot(vh, WV1))
    Z = _dot(jnp.concatenate([s, v], axis=1), M2) + b2
    vh = Z[:, 100:]
    ms = Z[:, :100] + _dot(_norm3(vh), S2v)
    mv = _dot(vh, WV2)
    return _pad12(jnp.concatenate([ms, mv], axis=1))


def _upd_math(tbl, p0, p1, inv, w):
    (n0g, n0b, F0, f0b, FS0v, FWV0, F1, f1b, FS1v, FWV1, n1g, n1b) = w
    agg = (p0 + p1) * inv
    s = tbl[:, :100] + agg[:, :100]
    v = tbl[:, 100:148] + agg[:, 100:148]
    s, v = _ln_math(s, v, n0g, n0b)
    X = _dot(jnp.concatenate([s, v], axis=1), F0) + f0b
    vh = X[:, 400:]
    hs = jax.nn.relu(X[:, :400] + _dot(_norm3(vh), FS0v))
    hv = _gate(_dot(vh, FWV0))
    Y = _dot(jnp.concatenate([hs, hv], axis=1), F1) + f1b
    vh = Y[:, 100:]
    ds = Y[:, :100] + _dot(_norm3(vh), FS1v)
    dv = _dot(vh, FWV1)
    s, v = _ln_math(s + ds, v + dv, n1g, n1b)
    return _pad12(jnp.concatenate([s, v], axis=1))


def _mlp_math(sf, vf, w):
    (w0s, w0v, b0, w1, b1, w2, b2, u0, ub0, u1, ub1, u2s, u2v, ub2s, ub2v) = w
    h = jax.nn.relu(_dot(sf, w0s) + _dot(vf, w0v) + b0)
    h = jax.nn.relu(_dot(h, w1) + b1)
    hs = _dot(h, w2) + b2
    h = jax.nn.relu(_dot(hs, u0) + ub0)
    h = jax.nn.relu(_dot(h, u1) + ub1)
    return hs, _dot(h, u2s) + ub2s, _dot(h, u2v) + ub2v


def _out_math(tbl, w):
    WH, WSs, WSv, b = w
    vh = _dot(tbl[:, 100:148], WH)
    return _dot(tbl[:, :100], WSs) + _dot(_norm3(vh), WSv) + b


def _wspecs(wts):
    return [pl.BlockSpec(w.shape, (lambda i, nd=w.ndim: (0,) * nd)) for w in wts]


def _tc_embed(s6, v9, cnt, wts):
    bn = 320
    nblk = _N // bn

    def body(*refs):
        s6r, v9r, cr = refs[0], refs[1], refs[2]
        wr = tuple(r[...] for r in refs[3:-2])
        c = cr[...]
        tbl, inv = _embed_math(s6r[...], v9r[...], c[0], c[1], wr)
        refs[-2][...] = tbl
        refs[-1][...] = inv

    return pl.pallas_call(
        body,
        grid=(nblk,),
        in_specs=[pl.BlockSpec((bn, 6), lambda i: (i, 0)),
                  pl.BlockSpec((bn, 9), lambda i: (i, 0)),
                  pl.BlockSpec((2, bn, 16), lambda i: (0, i, 0))] + _wspecs(wts),
        out_specs=[pl.BlockSpec((bn, _W), lambda i: (i, 0)),
                   pl.BlockSpec((bn, 1), lambda i: (i, 0))],
        out_shape=[jax.ShapeDtypeStruct((_N, _W), _F32),
                   jax.ShapeDtypeStruct((_N, 1), _F32)],
    )(s6, v9, cnt, *wts)


def _tc_edge(es, ev, wts):
    be = 1024
    nblk = _E // be

    def body(*refs):
        wr = tuple(r[...] for r in refs[2:-1])
        refs[-1][...] = _edge_math(refs[0][...], refs[1][...], wr)

    return pl.pallas_call(
        body,
        grid=(nblk,),
        in_specs=[pl.BlockSpec((be, 32), lambda i: (i, 0)),
                  pl.BlockSpec((be, 3), lambda i: (i, 0))] + _wspecs(wts),
        out_specs=pl.BlockSpec((be, _EF), lambda i: (i, 0)),
        out_shape=jax.ShapeDtypeStruct((_E, _EF), _F32),
    )(es, ev, *wts)


def _tc_message(g, ef, wts):
    be = 512
    nblk = _E // be

    def body(*refs):
        wr = tuple(r[...] for r in refs[3:-1])
        refs[-1][...] = _msg_math(refs[0][...], refs[1][...], refs[2][...], wr)

    return pl.pallas_call(
        body,
        grid=(nblk,),
        in_specs=[pl.BlockSpec((be, _W), lambda i: (i, 0)),
                  pl.BlockSpec((be, _W), lambda i, nb=nblk: (i + nb, 0)),
                  pl.BlockSpec((be, _EF), lambda i: (i, 0))] + _wspecs(wts),
        out_specs=pl.BlockSpec((be, _W), lambda i: (i, 0)),
        out_shape=jax.ShapeDtypeStruct((_E, _W), _F32),
    )(g, g, ef, *wts)


def _tc_update(tbl, parts, inv, wts):
    bn = 320
    nblk = _N // bn

    def body(*refs):
        wr = tuple(r[...] for r in refs[3:-1])
        p = refs[1][...]
        refs[-1][...] = _upd_math(refs[0][...], p[0], p[1], refs[2][...], wr)

    return pl.pallas_call(
        body,
        grid=(nblk,),
        in_specs=[pl.BlockSpec((bn, _W), lambda i: (i, 0)),
                  pl.BlockSpec((2, bn, _W), lambda i: (0, i, 0)),
                  pl.BlockSpec((bn, 1), lambda i: (i, 0))] + _wspecs(wts),
        out_specs=pl.BlockSpec((bn, _W), lambda i: (i, 0)),
        out_shape=jax.ShapeDtypeStruct((_N, _W), _F32),
    )(tbl, parts, inv, *wts)


def _tc_mlp(sf, vf, wts):
    def body(*refs):
        wr = tuple(r[...] for r in refs[2:-3])
        hs, os_, ov = _mlp_math(refs[0][...], refs[1][...], wr)
        refs[-3][...] = hs
        refs[-2][...] = os_
        refs[-1][...] = ov

    return pl.pallas_call(
        body,
        out_shape=[jax.ShapeDtypeStruct((_B, 16), _F32),
                   jax.ShapeDtypeStruct((_B, 5000), _F32),
                   jax.ShapeDtypeStruct((_B, 2400), _F32)],
    )(sf, vf, *wts)


def _tc_out(tbl, wts):
    bn = 320
    nblk = _N // bn

    def body(*refs):
        wr = tuple(r[...] for r in refs[1:-1])
        refs[-1][...] = _out_math(refs[0][...], wr)

    return pl.pallas_call(
        body,
        grid=(nblk,),
        in_specs=[pl.BlockSpec((bn, _W), lambda i: (i, 0))] + _wspecs(wts),
        out_specs=pl.BlockSpec((bn, 3), lambda i: (i, 0)),
        out_shape=jax.ShapeDtypeStruct((_N, 3), _F32),
    )(tbl, *wts)


def _sc_mesh():
    return plsc.VectorSubcoreMesh(core_axis_name="core", subcore_axis_name="subcore")


_SC_PARAMS = pltpu.CompilerParams(use_tc_tiling_on_sc=False)


def _sc_gather(table, idx2):
    ni = idx2.shape[1]

    @functools.partial(
        pl.kernel,
        out_type=jax.ShapeDtypeStruct((ni, _W), _F32),
        mesh=_sc_mesh(),
        compiler_params=_SC_PARAMS,
    )
    def k(x_hbm, i_hbm, o_hbm):
        def body(i_vmem, o_vmem):
            pltpu.sync_copy(x_hbm.at[i_vmem.at[0]], o_vmem)

        pltpu.emit_pipeline(
            body,
            grid=(ni // _GW,),
            in_specs=[pl.BlockSpec((1, _GW), lambda i: (0, i))],
            out_specs=[pl.BlockSpec((_GW, _W), lambda i: (i, 0))],
            core_axis_name=("core", "subcore"),
            dimension_semantics=(pltpu.PARALLEL,),
        )(i_hbm, o_hbm)

    return k(table, idx2)


def _sc_scatter(msg, idx2, zeros):
    @functools.partial(
        pl.kernel,
        out_type=jax.ShapeDtypeStruct((2, _N, _W), _F32),
        mesh=_sc_mesh(),
        scratch_types=[pltpu.VMEM_SHARED((_N, _W), _F32)],
        compiler_params=_SC_PARAMS,
    )
    def k(m_hbm, i_hbm, z_hbm, o_hbm, acc):
        cid = lax.axis_index("core")
        sid = lax.axis_index("subcore")
        rows = _N // 16
        sl = pl.ds(sid * rows, rows)
        pltpu.sync_copy(z_hbm.at[sl], acc.at[sl])
        plsc.subcore_barrier()

        def body(m_vmem, i_vmem):
            pltpu.sync_copy(m_vmem, acc.at[i_vmem.at[0]], add=True)

        pltpu.emit_pipeline(
            body,
            grid=(_E // _SW,),
            in_specs=[pl.BlockSpec((_SW, _W), lambda i: (i, 0)),
                      pl.BlockSpec((1, _SW), lambda i: (0, i))],
            out_specs=[],
            core_axis_name=("core", "subcore"),
            dimension_semantics=(pltpu.PARALLEL,),
        )(m_hbm, i_hbm)
        plsc.subcore_barrier()
        pltpu.sync_copy(acc.at[sl], o_hbm.at[cid, sl])

    return k(msg, idx2, zeros)


def _sc_counts(idx, ones, zeros16):
    @functools.partial(
        pl.kernel,
        out_type=jax.ShapeDtypeStruct((2, _N, 16), _F32),
        mesh=_sc_mesh(),
        scratch_types=[pltpu.VMEM_SHARED((_N, 16), _F32),
                       pltpu.VMEM((_CW, 16), _F32),
                       pltpu.VMEM((_CW,), jnp.int32)],
        compiler_params=_SC_PARAMS,
    )
    def k(i_hbm, ones_hbm, z_hbm, o_hbm, acc, ones_v, idx_v):
        cid = lax.axis_index("core")
        sid = lax.axis_index("subcore")
        rows = _N // 16
        sl = pl.ds(sid * rows, rows)
        pltpu.sync_copy(z_hbm.at[sl], acc.at[sl])
        pltpu.sync_copy(ones_hbm, ones_v)
        plsc.subcore_barrier()
        per_w = _E // 32
        base = (sid * 2 + cid) * per_w

        @pl.loop(0, per_w // _CW)
        def _(j):
            pltpu.sync_copy(i_hbm.at[pl.ds(base + j * _CW, _CW)], idx_v)
            pltpu.sync_copy(ones_v, acc.at[idx_v], add=True)

        plsc.subcore_barrier()
        pltpu.sync_copy(acc.at[sl], o_hbm.at[cid, sl])

    return k(idx, ones, zeros16)


def kernel(h_V_s, h_V_v, edge_index, h_E_s, h_E_v, params):
    s6 = h_V_s.reshape(-1, 6)
    v9 = jnp.swapaxes(h_V_v.reshape(-1, 3, 3), 1, 2).reshape(-1, 9)
    es0 = h_E_s.reshape(-1, 32)
    ev0 = h_E_v.reshape(-1, 3)
    idx_all = edge_index.reshape(1, 2 * _E)
    dst2 = edge_index[1].reshape(1, _E)
    dst1 = edge_index[1]

    zeros = jnp.zeros((_N, _W), _F32)
    zeros16 = jnp.zeros((_N, 16), _F32)
    ones = jnp.ones((_CW, 16), _F32)

    wts = _prep(params)

    cnt = _sc_counts(dst1, ones, zeros16)
    table, inv = _tc_embed(s6, v9, cnt, wts["W_v"])
    ef = _tc_edge(es0, ev0, wts["W_e"])

    def conv_layer(table, lw):
        g = _sc_gather(table, idx_all)
        msg = _tc_message(g, ef, lw["msg"])
        parts = _sc_scatter(msg, dst2, zeros)
        return _tc_update(table, parts, inv, lw["ff"])

    for lw in wts["enc"]:
        table = conv_layer(table, lw)

    sf = table[:, :100].reshape(_B, _NODE * 100)
    vf = table[:, 100:148].reshape(_B, _NODE * 48)
    h_small, os_, ov = _tc_mlp(sf, vf, wts["mlp"])
    table = jnp.concatenate(
        [os_.reshape(_N, 100), ov.reshape(_N, 48), jnp.zeros((_N, 12), _F32)],
        axis=1)

    for lw in wts["dec"]:
        table = conv_layer(table, lw)

    pred = _tc_out(table, wts["W_out"])
    return pred.reshape(_B, _NODE, 3), h_small


# speedup vs baseline: 14.0376x; 1.2829x over previous
import functools

import numpy as np
import jax
import jax.numpy as jnp
from jax import lax
from jax.experimental import pallas as pl
from jax.experimental.pallas import tpu as pltpu
from jax.experimental.pallas import tpu_sc as plsc

_N = 3200
_E = 51200
_B = 64
_NODE = 50
_W = 256
_WS = 160
_EF = 35
_F32 = jnp.float32

_GW = 128
_SW = 80
_CW = 80


def _bd3(w):
    return jnp.kron(jnp.eye(3, dtype=w.dtype), w)


def _gvp_mats(p):
    wh = p["wh"]
    ws = p["ws_w"]
    h = wh.shape[1]
    si = ws.shape[0] - h
    out = {
        "WH": _bd3(wh),
        "WSs": ws[:si],
        "WSv": ws[si:],
        "b": p["ws_b"][None, :],
    }
    if "wv" in p:
        out["WV"] = _bd3(p["wv"])
    return out


def _msg_prep(p):
    g0 = _gvp_mats(p["msg0"])
    rs = np.array([a * 33 + c for a in range(3) for c in range(16)])
    re = np.array([a * 33 + 16 for a in range(3)])
    rd = np.array([a * 33 + 17 + c for a in range(3) for c in range(16)])
    A_src = (jnp.zeros((_W, 199), _F32)
             .at[0:100, 0:100].set(g0["WSs"][0:100])
             .at[100:148, 100:199].set(g0["WH"][rs]))
    A_dst = (jnp.zeros((_W, 199), _F32)
             .at[0:100, 0:100].set(g0["WSs"][132:232])
             .at[100:148, 100:199].set(g0["WH"][rd]))
    A_e = (jnp.zeros((_EF, 199), _F32)
           .at[0:32, 0:100].set(g0["WSs"][100:132])
           .at[32:35, 100:199].set(g0["WH"][re]))
    b0 = jnp.zeros((1, 199), _F32).at[:, 0:100].set(g0["b"])

    def mid(gk):
        M = (jnp.zeros((148, 148), _F32)
             .at[0:100, 0:100].set(gk["WSs"])
             .at[100:148, 100:148].set(gk["WH"]))
        b = jnp.zeros((1, 148), _F32).at[:, 0:100].set(gk["b"])
        return M, b

    g1 = _gvp_mats(p["msg1"])
    g2 = _gvp_mats(p["msg2"])
    M1, b1 = mid(g1)
    M2, b2 = mid(g2)
    return (A_src, A_dst, A_e, b0, g0["WSv"], g0["WV"],
            M1, b1, g1["WSv"], g1["WV"],
            M2, b2, g2["WSv"], g2["WV"])


def _ff_prep(p):
    f0 = _gvp_mats(p["ff0"])
    f1 = _gvp_mats(p["ff1"])
    F0 = (jnp.zeros((148, 496), _F32)
          .at[0:100, 0:400].set(f0["WSs"])
          .at[100:148, 400:496].set(f0["WH"]))
    f0b = jnp.zeros((1, 496), _F32).at[:, 0:400].set(f0["b"])
    F1 = (jnp.zeros((496, 196), _F32)
          .at[0:400, 0:100].set(f1["WSs"])
          .at[400:496, 100:196].set(f1["WH"]))
    f1b = jnp.zeros((1, 196), _F32).at[:, 0:100].set(f1["b"])
    n0, n1 = p["norm0"], p["norm1"]
    return (n0["g"][None, :], n0["b"][None, :], F0, f0b, f0["WSv"], f0["WV"],
            F1, f1b, f1["WSv"], f1["WV"], n1["g"][None, :], n1["b"][None, :])


def _prep(params):
    wv = _gvp_mats(params["W_v"])
    wvln = params["W_v_ln"]
    we = params["W_e"]
    weln = params["W_e_ln"]
    wo = _gvp_mats(params["W_out"])

    perm = np.array([n * 48 + c * 3 + a
                     for n in range(_NODE) for a in range(3) for c in range(16)])
    (q0w, q0b), (q1w, q1b), (q2w, q2b) = params["sq"]
    (u0w, u0b), (u1w, u1b), (u2w, u2b) = params["us"]
    mlp = (q0w[:5000], q0w[5000:][perm], q0b[None, :],
           q1w, q1b[None, :], q2w, q2b[None, :],
           u0w, u0b[None, :], u1w, u1b[None, :],
           u2w[:, :5000], u2w[:, 5000:][:, perm],
           u2b[None, :5000], u2b[None, 5000:][:, perm])

    return {
        "W_v": (wv["WH"], wv["WSs"], wv["WSv"], wv["b"], wv["WV"],
                wvln["g"][None, :], wvln["b"][None, :]),
        "W_e": (we["wh"], we["ws_w"][:32], we["ws_w"][32:33], we["ws_b"][None, :],
                we["wv"], weln["g"][None, :], weln["b"][None, :]),
        "enc": [{"msg": _msg_prep(p), "ff": _ff_prep(p)} for p in params["enc"]],
        "dec": [{"msg": _msg_prep(p), "ff": _ff_prep(p)} for p in params["dec"]],
        "mlp": mlp,
        "W_out": (wo["WH"], wo["WSs"], wo["WSv"], wo["b"]),
    }


def _dot(a, b):
    return jnp.dot(a, b, preferred_element_type=_F32)


def _norm3(x):
    h = x.shape[-1] // 3
    sq = x[:, :h] * x[:, :h] + x[:, h:2 * h] * x[:, h:2 * h] + x[:, 2 * h:] * x[:, 2 * h:]
    return jnp.sqrt(jnp.maximum(sq, 1e-8))


def _gate(v):
    g = jax.nn.sigmoid(_norm3(v))
    return v * jnp.concatenate([g, g, g], axis=1)


def _ln_math(s, v, g, b):
    mu = jnp.mean(s, axis=-1, keepdims=True)
    var = jnp.mean((s - mu) ** 2, axis=-1, keepdims=True)
    s = (s - mu) / jnp.sqrt(var + 1e-5) * g + b
    c = v.shape[-1] // 3
    sq = v[:, :c] * v[:, :c] + v[:, c:2 * c] * v[:, c:2 * c] + v[:, 2 * c:] * v[:, 2 * c:]
    vn = jnp.sqrt(jnp.mean(jnp.maximum(sq, 1e-8), axis=-1, keepdims=True))
    return s, v / vn


def _padw(x, w=_W):
    return jnp.concatenate([x, jnp.zeros((x.shape[0], w - 148), _F32)], axis=1)


def _embed_math(s6, v9, c0, c1, w):
    WH, WSs, WSv, b, WV, lg, lb = w
    vh = _dot(v9, WH)
    s = _dot(s6, WSs) + _dot(_norm3(vh), WSv) + b
    v = _dot(vh, WV)
    s, v = _ln_math(s, v, lg, lb)
    inv = 1.0 / jnp.maximum(c0[:, 0:1] + c1[:, 0:1], 1.0)
    return _padw(jnp.concatenate([s, v], axis=1)), inv


def _edge_math(es, ev, w):
    whs, WSs, wsv, b, wvs, lg, lb = w
    vh = ev * whs[0, 0]
    vn = jnp.sqrt(jnp.maximum(jnp.sum(vh * vh, axis=1, keepdims=True), 1e-8))
    s = _dot(es, WSs) + vn * wsv + b
    vo = vh * wvs[0, 0]
    mu = jnp.mean(s, axis=-1, keepdims=True)
    var = jnp.mean((s - mu) ** 2, axis=-1, keepdims=True)
    s = (s - mu) / jnp.sqrt(var + 1e-5) * lg + lb
    sq = jnp.sum(vo * vo, axis=1, keepdims=True)
    vo = vo / jnp.sqrt(jnp.maximum(sq, 1e-8))
    return jnp.concatenate([s, vo], axis=1)


def _msg_math(gs, gd, ef, w):
    (A_src, A_dst, A_e, b0, S0v, WV0, M1, b1, S1v, WV1, M2, b2, S2v, WV2) = w
    X = _dot(gs, A_src) + _dot(gd, A_dst) + _dot(ef, A_e) + b0
    vh = X[:, 100:]
    s = jax.nn.relu(X[:, :100] + _dot(_norm3(vh), S0v))
    v = _gate(_dot(vh, WV0))
    Y = _dot(jnp.concatenate([s, v], axis=1), M1) + b1
    vh = Y[:, 100:]
    s = jax.nn.relu(Y[:, :100] + _dot(_norm3(vh), S1v))
    v = _gate(_dot(vh, WV1))
    Z = _dot(jnp.concatenate([s, v], axis=1), M2) + b2
    vh = Z[:, 100:]
    ms = Z[:, :100] + _dot(_norm3(vh), S2v)
    mv = _dot(vh, WV2)
    return _padw(jnp.concatenate([ms, mv], axis=1), _WS)


def _upd_math(tbl, p0, p1, inv, w):
    (n0g, n0b, F0, f0b, FS0v, FWV0, F1, f1b, FS1v, FWV1, n1g, n1b) = w
    agg = (p0 + p1) * inv
    s = tbl[:, :100] + agg[:, :100]
    v = tbl[:, 100:148] + agg[:, 100:148]
    s, v = _ln_math(s, v, n0g, n0b)
    X = _dot(jnp.concatenate([s, v], axis=1), F0) + f0b
    vh = X[:, 400:]
    hs = jax.nn.relu(X[:, :400] + _dot(_norm3(vh), FS0v))
    hv = _gate(_dot(vh, FWV0))
    Y = _dot(jnp.concatenate([hs, hv], axis=1), F1) + f1b
    vh = Y[:, 100:]
    ds = Y[:, :100] + _dot(_norm3(vh), FS1v)
    dv = _dot(vh, FWV1)
    s, v = _ln_math(s + ds, v + dv, n1g, n1b)
    return _padw(jnp.concatenate([s, v], axis=1))


def _mlp_math(sf, vf, w):
    (w0s, w0v, b0, w1, b1, w2, b2, u0, ub0, u1, ub1, u2s, u2v, ub2s, ub2v) = w
    h = jax.nn.relu(_dot(sf, w0s) + _dot(vf, w0v) + b0)
    h = jax.nn.relu(_dot(h, w1) + b1)
    hs = _dot(h, w2) + b2
    h = jax.nn.relu(_dot(hs, u0) + ub0)
    h = jax.nn.relu(_dot(h, u1) + ub1)
    return hs, _dot(h, u2s) + ub2s, _dot(h, u2v) + ub2v


def _out_math(tbl, w):
    WH, WSs, WSv, b = w
    vh = _dot(tbl[:, 100:148], WH)
    return _dot(tbl[:, :100], WSs) + _dot(_norm3(vh), WSv) + b


def _wspecs(wts):
    return [pl.BlockSpec(w.shape, (lambda i, nd=w.ndim: (0,) * nd)) for w in wts]


def _tc_embed(s6, v9, cnt, wts):
    bn = 320
    nblk = _N // bn

    def body(*refs):
        s6r, v9r, cr = refs[0], refs[1], refs[2]
        wr = tuple(r[...] for r in refs[3:-2])
        c = cr[...]
        tbl, inv = _embed_math(s6r[...], v9r[...], c[0], c[1], wr)
        refs[-2][...] = tbl
        refs[-1][...] = inv

    return pl.pallas_call(
        body,
        grid=(nblk,),
        in_specs=[pl.BlockSpec((bn, 6), lambda i: (i, 0)),
                  pl.BlockSpec((bn, 9), lambda i: (i, 0)),
                  pl.BlockSpec((2, bn, 16), lambda i: (0, i, 0))] + _wspecs(wts),
        out_specs=[pl.BlockSpec((bn, _W), lambda i: (i, 0)),
                   pl.BlockSpec((bn, 1), lambda i: (i, 0))],
        out_shape=[jax.ShapeDtypeStruct((_N, _W), _F32),
                   jax.ShapeDtypeStruct((_N, 1), _F32)],
    )(s6, v9, cnt, *wts)


def _tc_edge(es, ev, wts):
    be = 1024
    nblk = _E // be

    def body(*refs):
        wr = tuple(r[...] for r in refs[2:-1])
        refs[-1][...] = _edge_math(refs[0][...], refs[1][...], wr)

    return pl.pallas_call(
        body,
        grid=(nblk,),
        in_specs=[pl.BlockSpec((be, 32), lambda i: (i, 0)),
                  pl.BlockSpec((be, 3), lambda i: (i, 0))] + _wspecs(wts),
        out_specs=pl.BlockSpec((be, _EF), lambda i: (i, 0)),
        out_shape=jax.ShapeDtypeStruct((_E, _EF), _F32),
    )(es, ev, *wts)


def _tc_message(g, ef, wts):
    be = 1024
    nblk = _E // be

    def body(*refs):
        wr = tuple(r[...] for r in refs[3:-1])
        refs[-1][...] = _msg_math(refs[0][...], refs[1][...], refs[2][...], wr)

    return pl.pallas_call(
        body,
        grid=(nblk,),
        in_specs=[pl.BlockSpec((be, _W), lambda i: (i, 0)),
                  pl.BlockSpec((be, _W), lambda i, nb=nblk: (i + nb, 0)),
                  pl.BlockSpec((be, _EF), lambda i: (i, 0))] + _wspecs(wts),
        out_specs=pl.BlockSpec((be, _WS), lambda i: (i, 0)),
        out_shape=jax.ShapeDtypeStruct((_E, _WS), _F32),
    )(g, g, ef, *wts)


def _tc_update(tbl, parts, inv, wts):
    bn = 320
    nblk = _N // bn

    def body(*refs):
        wr = tuple(r[...] for r in refs[3:-1])
        p = refs[1][...]
        refs[-1][...] = _upd_math(refs[0][...], p[0], p[1], refs[2][...], wr)

    return pl.pallas_call(
        body,
        grid=(nblk,),
        in_specs=[pl.BlockSpec((bn, _W), lambda i: (i, 0)),
                  pl.BlockSpec((2, bn, _WS), lambda i: (0, i, 0)),
                  pl.BlockSpec((bn, 1), lambda i: (i, 0))] + _wspecs(wts),
        out_specs=pl.BlockSpec((bn, _W), lambda i: (i, 0)),
        out_shape=jax.ShapeDtypeStruct((_N, _W), _F32),
    )(tbl, parts, inv, *wts)


def _tc_mlp(sf, vf, wts):
    def body(*refs):
        wr = tuple(r[...] for r in refs[2:-3])
        hs, os_, ov = _mlp_math(refs[0][...], refs[1][...], wr)
        refs[-3][...] = hs
        refs[-2][...] = os_
        refs[-1][...] = ov

    return pl.pallas_call(
        body,
        out_shape=[jax.ShapeDtypeStruct((_B, 16), _F32),
                   jax.ShapeDtypeStruct((_B, 5000), _F32),
                   jax.ShapeDtypeStruct((_B, 2400), _F32)],
    )(sf, vf, *wts)


def _tc_out(tbl, wts):
    bn = 320
    nblk = _N // bn

    def body(*refs):
        wr = tuple(r[...] for r in refs[1:-1])
        refs[-1][...] = _out_math(refs[0][...], wr)

    return pl.pallas_call(
        body,
        grid=(nblk,),
        in_specs=[pl.BlockSpec((bn, _W), lambda i: (i, 0))] + _wspecs(wts),
        out_specs=pl.BlockSpec((bn, 3), lambda i: (i, 0)),
        out_shape=jax.ShapeDtypeStruct((_N, 3), _F32),
    )(tbl, *wts)


def _sc_mesh():
    return plsc.VectorSubcoreMesh(core_axis_name="core", subcore_axis_name="subcore")


_SC_PARAMS = pltpu.CompilerParams(use_tc_tiling_on_sc=False)


def _sc_gather(table, idx2):
    ni = idx2.shape[1]

    @functools.partial(
        pl.kernel,
        out_type=jax.ShapeDtypeStruct((ni, _W), _F32),
        mesh=_sc_mesh(),
    )
    def k(x_hbm, i_hbm, o_hbm):
        def body(i_vmem, o_vmem):
            pltpu.sync_copy(x_hbm.at[i_vmem.at[0]], o_vmem)

        pltpu.emit_pipeline(
            body,
            grid=(ni // _GW,),
            in_specs=[pl.BlockSpec((1, _GW), lambda i: (0, i))],
            out_specs=[pl.BlockSpec((_GW, _W), lambda i: (i, 0))],
            core_axis_name=("core", "subcore"),
            dimension_semantics=(pltpu.PARALLEL,),
        )(i_hbm, o_hbm)

    return k(table, idx2)


def _sc_scatter(msg, idx2, zeros):
    @functools.partial(
        pl.kernel,
        out_type=jax.ShapeDtypeStruct((2, _N, _WS), _F32),
        mesh=_sc_mesh(),
        scratch_types=[pltpu.VMEM_SHARED((_N, _WS), _F32)],
        compiler_params=_SC_PARAMS,
    )
    def k(m_hbm, i_hbm, z_hbm, o_hbm, acc):
        cid = lax.axis_index("core")
        sid = lax.axis_index("subcore")
        rows = _N // 16
        sl = pl.ds(sid * rows, rows)
        pltpu.sync_copy(z_hbm.at[sl], acc.at[sl])
        plsc.subcore_barrier()

        def body(m_vmem, i_vmem):
            pltpu.sync_copy(m_vmem, acc.at[i_vmem.at[0]], add=True)

        pltpu.emit_pipeline(
            body,
            grid=(_E // _SW,),
            in_specs=[pl.BlockSpec((_SW, _WS), lambda i: (i, 0)),
                      pl.BlockSpec((1, _SW), lambda i: (0, i))],
            out_specs=[],
            core_axis_name=("core", "subcore"),
            dimension_semantics=(pltpu.PARALLEL,),
        )(m_hbm, i_hbm)
        plsc.subcore_barrier()
        pltpu.sync_copy(acc.at[sl], o_hbm.at[cid, sl])

    return k(msg, idx2, zeros)


def _sc_counts(idx, ones, zeros16):
    @functools.partial(
        pl.kernel,
        out_type=jax.ShapeDtypeStruct((2, _N, 16), _F32),
        mesh=_sc_mesh(),
        scratch_types=[pltpu.VMEM_SHARED((_N, 16), _F32),
                       pltpu.VMEM((_CW, 16), _F32),
                       pltpu.VMEM((_CW,), jnp.int32)],
        compiler_params=_SC_PARAMS,
    )
    def k(i_hbm, ones_hbm, z_hbm, o_hbm, acc, ones_v, idx_v):
        cid = lax.axis_index("core")
        sid = lax.axis_index("subcore")
        rows = _N // 16
        sl = pl.ds(sid * rows, rows)
        pltpu.sync_copy(z_hbm.at[sl], acc.at[sl])
        pltpu.sync_copy(ones_hbm, ones_v)
        plsc.subcore_barrier()
        per_w = _E // 32
        base = (sid * 2 + cid) * per_w

        @pl.loop(0, per_w // _CW)
        def _(j):
            pltpu.sync_copy(i_hbm.at[pl.ds(base + j * _CW, _CW)], idx_v)
            pltpu.sync_copy(ones_v, acc.at[idx_v], add=True)

        plsc.subcore_barrier()
        pltpu.sync_copy(acc.at[sl], o_hbm.at[cid, sl])

    return k(idx, ones, zeros16)


def kernel(h_V_s, h_V_v, edge_index, h_E_s, h_E_v, params):
    s6 = h_V_s.reshape(-1, 6)
    v9 = jnp.swapaxes(h_V_v.reshape(-1, 3, 3), 1, 2).reshape(-1, 9)
    es0 = h_E_s.reshape(-1, 32)
    ev0 = h_E_v.reshape(-1, 3)
    idx_all = edge_index.reshape(1, 2 * _E)
    dst2 = edge_index[1].reshape(1, _E)
    dst1 = edge_index[1]

    zeros = jnp.zeros((_N, _WS), _F32)
    zeros16 = jnp.zeros((_N, 16), _F32)
    ones = jnp.ones((_CW, 16), _F32)

    wts = _prep(params)

    cnt = _sc_counts(dst1, ones, zeros16)
    table, inv = _tc_embed(s6, v9, cnt, wts["W_v"])
    ef = _tc_edge(es0, ev0, wts["W_e"])

    def conv_layer(table, lw):
        g = _sc_gather(table, idx_all)
        msg = _tc_message(g, ef, lw["msg"])
        parts = _sc_scatter(msg, dst2, zeros)
        return _tc_update(table, parts, inv, lw["ff"])

    for lw in wts["enc"]:
        table = conv_layer(table, lw)

    sf = table[:, :100].reshape(_B, _NODE * 100)
    vf = table[:, 100:148].reshape(_B, _NODE * 48)
    h_small, os_, ov = _tc_mlp(sf, vf, wts["mlp"])
    table = jnp.concatenate(
        [os_.reshape(_N, 100), ov.reshape(_N, 48), jnp.zeros((_N, _W - 148), _F32)],
        axis=1)

    for lw in wts["dec"]:
        table = conv_layer(table, lw)

    pred = _tc_out(table, wts["W_out"])
    return pred.reshape(_B, _NODE, 3), h_small
